# Initial kernel scaffold; baseline (speedup 1.0000x reference)
import jax
import jax.numpy as jnp
from jax.experimental import pallas as pl


def kernel(q_points, s_points, s_feats, neighbor_indices, kernel_points, W1, gamma1, beta1, conv_W, gamma_c, beta_c, W2, gamma2, beta2):
    raise NotImplementedError("write your pallas kernel here")


# speedup vs baseline: 3.5519x; 3.5519x over previous
import functools

import jax
import jax.numpy as jnp
from jax import lax
from jax.experimental import pallas as pl
from jax.experimental.pallas import tpu as pltpu
from jax.experimental.pallas import tpu_sc as plsc

N = 10000
K = 32
C_IN = 128
C_OUT = 128
C_MID = 32
KP = 15
SIGMA = 2.0
GROUPS = 8
EPS = 1e-5
NEG_SLOPE = 0.1

D_TAB = 48
BM = 200


def _leaky(x):
    return jnp.where(x >= 0, x, NEG_SLOPE * x)


def _group_indicator(c, dtype):
    per_g = c // GROUPS
    row_g = lax.broadcasted_iota(jnp.int32, (c, GROUPS), 0) // per_g
    col = lax.broadcasted_iota(jnp.int32, (c, GROUPS), 1)
    return (row_g == col).astype(dtype)


def _group_norm(x, gamma, beta, n_rows):
    c = x.shape[1]
    ind = _group_indicator(c, x.dtype)
    cs = jnp.sum(x, axis=0, keepdims=True)
    css = jnp.sum(x * x, axis=0, keepdims=True)
    cnt = float(n_rows * (c // GROUPS))
    gmean = jnp.dot(cs, ind, preferred_element_type=jnp.float32) / cnt
    gmsq = jnp.dot(css, ind, preferred_element_type=jnp.float32) / cnt
    var = gmsq - gmean * gmean
    rstd = lax.rsqrt(var + EPS)
    mean_c = jnp.dot(gmean, ind.T, preferred_element_type=jnp.float32)
    rstd_c = jnp.dot(rstd, ind.T, preferred_element_type=jnp.float32)
    return (x - mean_c) * rstd_c * gamma + beta


def _unary1_body(x_ref, w_ref, g_ref, b_ref, o_ref):
    xm = jnp.dot(x_ref[...], w_ref[...], preferred_element_type=jnp.float32)
    o_ref[...] = _leaky(_group_norm(xm, g_ref[...], b_ref[...], N))


def _unary1(s_feats, W1, gamma1, beta1):
    return pl.pallas_call(
        _unary1_body,
        out_shape=jax.ShapeDtypeStruct((N, C_MID), jnp.float32),
    )(s_feats, W1, gamma1.reshape(1, C_MID), beta1.reshape(1, C_MID))


def _sc_gather(table, idx_flat):
    info = plsc.get_sparse_core_info()
    nw = info.num_cores * info.num_subcores
    tot = N * K
    per_w = tot // nw
    ch = 1000
    n_ch = per_w // ch
    mesh = plsc.VectorSubcoreMesh(core_axis_name="c", subcore_axis_name="s")

    @functools.partial(
        pl.kernel,
        mesh=mesh,
        compiler_params=pltpu.CompilerParams(use_tc_tiling_on_sc=False),
        out_type=jax.ShapeDtypeStruct((tot, D_TAB), jnp.float32),
        scratch_types=[
            pltpu.VMEM((ch,), jnp.int32),
            pltpu.VMEM((ch, D_TAB), jnp.float32),
            pltpu.SemaphoreType.DMA,
        ],
    )
    def gather(table_hbm, idx_hbm, out_hbm, idx_v, rows_v, sem):
        wid = lax.axis_index("s") * info.num_cores + lax.axis_index("c")
        base = wid * per_w

        def body(j, carry):
            off = base + j * ch
            pltpu.sync_copy(idx_hbm.at[pl.ds(off, ch)], idx_v)
            pltpu.async_copy(table_hbm.at[idx_v], rows_v, sem).wait()
            pltpu.sync_copy(rows_v, out_hbm.at[pl.ds(off, ch)])
            return carry

        lax.fori_loop(0, n_ch, body, 0)

    return gather(table, idx_flat)


def _kpconv_body(g_ref, q_ref, kpt_ref, r_ref, t_ref, w_ref, o_ref):
    q = q_ref[...]
    kpt = kpt_ref[...]
    rmat = r_ref[...]
    tmat = t_ref[...]
    wf = jnp.zeros((BM, KP * C_MID), jnp.float32)
    cnt = jnp.zeros((BM, 1), jnp.float32)
    for k in range(K):
        gk = g_ref[k]
        nf_k = gk[:, 0:C_MID]
        pk = gk[:, C_MID:C_MID + 3]
        r = pk - q
        dx = r[:, 0:1] - kpt[0:1, :]
        dy = r[:, 1:2] - kpt[1:2, :]
        dz = r[:, 2:3] - kpt[2:3, :]
        d2 = dx * dx + dy * dy + dz * dz
        w = jnp.maximum(1.0 - jnp.sqrt(d2) * (1.0 / SIGMA), 0.0)
        wrep = jnp.dot(w, rmat, preferred_element_type=jnp.float32)
        nfrep = jnp.dot(nf_k, tmat, preferred_element_type=jnp.float32)
        wf = wf + wrep * nfrep
        rs = jnp.sum(nf_k, axis=1, keepdims=True)
        cnt = cnt + (rs != 0).astype(jnp.float32)
    out = jnp.dot(wf, w_ref[...], preferred_element_type=jnp.float32)
    o_ref[...] = out / jnp.maximum(cnt, 1.0)


def _kpconv(g3, q_points, kp_t, r_mat, t_mat, w_flat):
    n_blocks = N // BM
    return pl.pallas_call(
        _kpconv_body,
        grid=(n_blocks,),
        in_specs=[
            pl.BlockSpec((K, BM, D_TAB), lambda i: (0, i, 0)),
            pl.BlockSpec((BM, 3), lambda i: (i, 0)),
            pl.BlockSpec((3, KP), lambda i: (0, 0)),
            pl.BlockSpec((KP, KP * C_MID), lambda i: (0, 0)),
            pl.BlockSpec((C_MID, KP * C_MID), lambda i: (0, 0)),
            pl.BlockSpec((KP * C_MID, C_MID), lambda i: (0, 0)),
        ],
        out_specs=pl.BlockSpec((BM, C_MID), lambda i: (i, 0)),
        out_shape=jax.ShapeDtypeStruct((N, C_MID), jnp.float32),
    )(g3, q_points, kp_t, r_mat, t_mat, w_flat)


def _tail_body(x_ref, f_ref, w2_ref, gc_ref, bc_ref, g2_ref, b2_ref, o_ref):
    x = _leaky(_group_norm(x_ref[...], gc_ref[...], bc_ref[...], N))
    y = jnp.dot(x, w2_ref[...], preferred_element_type=jnp.float32)
    y = _group_norm(y, g2_ref[...], b2_ref[...], N)
    o_ref[...] = _leaky(y + f_ref[...])


def _tail(out1, s_feats, W2, gamma_c, beta_c, gamma2, beta2):
    return pl.pallas_call(
        _tail_body,
        out_shape=jax.ShapeDtypeStruct((N, C_OUT), jnp.float32),
    )(out1, s_feats, W2,
      gamma_c.reshape(1, C_MID), beta_c.reshape(1, C_MID),
      gamma2.reshape(1, C_OUT), beta2.reshape(1, C_OUT))


def kernel(q_points, s_points, s_feats, neighbor_indices, kernel_points,
           W1, gamma1, beta1, conv_W, gamma_c, beta_c, W2, gamma2, beta2):
    y1 = _unary1(s_feats, W1, gamma1, beta1)
    table = jnp.concatenate(
        [y1, s_points, jnp.zeros((N, D_TAB - C_MID - 3), jnp.float32)], axis=1)
    idx_flat = neighbor_indices.astype(jnp.int32).T.reshape(-1)
    g = _sc_gather(table, idx_flat)
    g3 = g.reshape(K, N, D_TAB)
    w_flat = conv_W.reshape(KP * C_MID, C_MID)
    kp_t = kernel_points.T
    j = jnp.arange(KP * C_MID)
    r_mat = (jnp.arange(KP)[:, None] == (j[None, :] // C_MID)).astype(jnp.float32)
    t_mat = (jnp.arange(C_MID)[:, None] == (j[None, :] % C_MID)).astype(jnp.float32)
    out1 = _kpconv(g3, q_points, kp_t, r_mat, t_mat, w_flat)
    return _tail(out1, s_feats, W2, gamma_c, beta_c, gamma2, beta2)


# speedup vs baseline: 3.6702x; 1.0333x over previous
import functools

import jax
import jax.numpy as jnp
from jax import lax
from jax.experimental import pallas as pl
from jax.experimental.pallas import tpu as pltpu
from jax.experimental.pallas import tpu_sc as plsc

N = 10000
K = 32
C_IN = 128
C_OUT = 128
C_MID = 32
KP = 15
SIGMA = 2.0
GROUPS = 8
EPS = 1e-5
NEG_SLOPE = 0.1

D_TAB = 48
BM = 1000


def _leaky(x):
    return jnp.where(x >= 0, x, NEG_SLOPE * x)


def _group_indicator(c, dtype):
    per_g = c // GROUPS
    row_g = lax.broadcasted_iota(jnp.int32, (c, GROUPS), 0) // per_g
    col = lax.broadcasted_iota(jnp.int32, (c, GROUPS), 1)
    return (row_g == col).astype(dtype)


def _group_norm(x, gamma, beta, n_rows):
    c = x.shape[1]
    ind = _group_indicator(c, x.dtype)
    cs = jnp.sum(x, axis=0, keepdims=True)
    css = jnp.sum(x * x, axis=0, keepdims=True)
    cnt = float(n_rows * (c // GROUPS))
    gmean = jnp.dot(cs, ind, preferred_element_type=jnp.float32) / cnt
    gmsq = jnp.dot(css, ind, preferred_element_type=jnp.float32) / cnt
    var = gmsq - gmean * gmean
    rstd = lax.rsqrt(var + EPS)
    mean_c = jnp.dot(gmean, ind.T, preferred_element_type=jnp.float32)
    rstd_c = jnp.dot(rstd, ind.T, preferred_element_type=jnp.float32)
    return (x - mean_c) * rstd_c * gamma + beta


def _unary1_body(x_ref, p_ref, w_ref, g_ref, b_ref, o_ref):
    xm = jnp.dot(x_ref[...], w_ref[...], preferred_element_type=jnp.float32)
    y1 = _leaky(_group_norm(xm, g_ref[...], b_ref[...], N))
    o_ref[:, 0:C_MID] = y1
    o_ref[:, C_MID:C_MID + 3] = p_ref[...]
    rs = jnp.sum(y1, axis=1, keepdims=True)
    o_ref[:, C_MID + 3:C_MID + 4] = (rs != 0).astype(jnp.float32)
    o_ref[:, C_MID + 4:D_TAB] = jnp.zeros((N, D_TAB - C_MID - 4), jnp.float32)


def _unary1(s_feats, s_points, W1, gamma1, beta1):
    return pl.pallas_call(
        _unary1_body,
        out_shape=jax.ShapeDtypeStruct((N, D_TAB), jnp.float32),
    )(s_feats, s_points, W1, gamma1.reshape(1, C_MID), beta1.reshape(1, C_MID))


def _sc_gather(table, idx_flat):
    info = plsc.get_sparse_core_info()
    nw = info.num_cores * info.num_subcores
    tot = N * K
    per_w = tot // nw
    ch = 1000
    n_ch = per_w // ch
    mesh = plsc.VectorSubcoreMesh(core_axis_name="c", subcore_axis_name="s")

    @functools.partial(
        pl.kernel,
        mesh=mesh,
        compiler_params=pltpu.CompilerParams(use_tc_tiling_on_sc=False),
        out_type=jax.ShapeDtypeStruct((K, N, D_TAB), jnp.float32),
        scratch_types=[
            pltpu.VMEM((ch,), jnp.int32),
            pltpu.VMEM((ch, D_TAB), jnp.float32),
            pltpu.SemaphoreType.DMA,
        ],
    )
    def gather(table_hbm, idx_hbm, out_hbm, idx_v, rows_v, sem):
        wid = lax.axis_index("s") * info.num_cores + lax.axis_index("c")
        base = wid * per_w

        def body(j, carry):
            off = base + j * ch
            pltpu.sync_copy(idx_hbm.at[pl.ds(off, ch)], idx_v)
            pltpu.async_copy(table_hbm.at[idx_v], rows_v, sem).wait()
            pltpu.sync_copy(rows_v, out_hbm.at[wid, pl.ds(j * ch, ch)])
            return carry

        lax.fori_loop(0, n_ch, body, 0)

    return gather(table, idx_flat)


def _kpconv_body(q_ref, kpt_ref, r_ref, t_ref, w_ref, g_hbm, o_ref, gbuf, sems):
    i = pl.program_id(0)
    nb = pl.num_programs(0)

    def win_copy(blk, slot):
        return pltpu.make_async_copy(
            g_hbm.at[:, pl.ds(blk * BM, BM), :], gbuf.at[slot], sems.at[slot])

    @pl.when(i == 0)
    def _():
        win_copy(0, 0).start()

    @pl.when(i + 1 < nb)
    def _():
        win_copy(i + 1, (i + 1) % 2).start()

    win_copy(i, i % 2).wait()
    g = gbuf[i % 2]

    q = q_ref[...]
    kpt = kpt_ref[...]
    rmat = r_ref[...]
    tmat = t_ref[...]
    wf = jnp.zeros((BM, KP * C_MID), jnp.float32)
    cnt = jnp.zeros((BM, 1), jnp.float32)
    for k in range(K):
        gk = g[k]
        nf_k = gk[:, 0:C_MID]
        pk = gk[:, C_MID:C_MID + 3]
        r = pk - q
        dx = r[:, 0:1] - kpt[0:1, :]
        dy = r[:, 1:2] - kpt[1:2, :]
        dz = r[:, 2:3] - kpt[2:3, :]
        d2 = dx * dx + dy * dy + dz * dz
        w = jnp.maximum(1.0 - jnp.sqrt(d2) * (1.0 / SIGMA), 0.0)
        wrep = jnp.dot(w, rmat, preferred_element_type=jnp.float32)
        nfrep = jnp.dot(nf_k, tmat, preferred_element_type=jnp.float32)
        wf = wf + wrep * nfrep
        cnt = cnt + gk[:, C_MID + 3:C_MID + 4]
    out = jnp.dot(wf, w_ref[...], preferred_element_type=jnp.float32)
    o_ref[...] = out / jnp.maximum(cnt, 1.0)


def _kpconv(g3, q_points, kp_t, r_mat, t_mat, w_flat):
    n_blocks = N // BM
    return pl.pallas_call(
        _kpconv_body,
        grid=(n_blocks,),
        in_specs=[
            pl.BlockSpec((BM, 3), lambda i: (i, 0)),
            pl.BlockSpec((3, KP), lambda i: (0, 0)),
            pl.BlockSpec((KP, KP * C_MID), lambda i: (0, 0)),
            pl.BlockSpec((C_MID, KP * C_MID), lambda i: (0, 0)),
            pl.BlockSpec((KP * C_MID, C_MID), lambda i: (0, 0)),
            pl.BlockSpec(memory_space=pl.ANY),
        ],
        out_specs=pl.BlockSpec((BM, C_MID), lambda i: (i, 0)),
        out_shape=jax.ShapeDtypeStruct((N, C_MID), jnp.float32),
        scratch_shapes=[
            pltpu.VMEM((2, K, BM, D_TAB), jnp.float32),
            pltpu.SemaphoreType.DMA((2,)),
        ],
    )(q_points, kp_t, r_mat, t_mat, w_flat, g3)


def _tail_body(x_ref, f_ref, w2_ref, gc_ref, bc_ref, g2_ref, b2_ref, o_ref):
    x = _leaky(_group_norm(x_ref[...], gc_ref[...], bc_ref[...], N))
    y = jnp.dot(x, w2_ref[...], preferred_element_type=jnp.float32)
    y = _group_norm(y, g2_ref[...], b2_ref[...], N)
    o_ref[...] = _leaky(y + f_ref[...])


def _tail(out1, s_feats, W2, gamma_c, beta_c, gamma2, beta2):
    return pl.pallas_call(
        _tail_body,
        out_shape=jax.ShapeDtypeStruct((N, C_OUT), jnp.float32),
    )(out1, s_feats, W2,
      gamma_c.reshape(1, C_MID), beta_c.reshape(1, C_MID),
      gamma2.reshape(1, C_OUT), beta2.reshape(1, C_OUT))


def kernel(q_points, s_points, s_feats, neighbor_indices, kernel_points,
           W1, gamma1, beta1, conv_W, gamma_c, beta_c, W2, gamma2, beta2):
    table = _unary1(s_feats, s_points, W1, gamma1, beta1)
    idx_flat = neighbor_indices.astype(jnp.int32).T.reshape(-1)
    g3 = _sc_gather(table, idx_flat)
    w_flat = conv_W.reshape(KP * C_MID, C_MID)
    kp_t = kernel_points.T
    j = jnp.arange(KP * C_MID)
    r_mat = (jnp.arange(KP)[:, None] == (j[None, :] // C_MID)).astype(jnp.float32)
    t_mat = (jnp.arange(C_MID)[:, None] == (j[None, :] % C_MID)).astype(jnp.float32)
    out1 = _kpconv(g3, q_points, kp_t, r_mat, t_mat, w_flat)
    return _tail(out1, s_feats, W2, gamma_c, beta_c, gamma2, beta2)


# speedup vs baseline: 3.8529x; 1.0498x over previous
import functools

import jax
import jax.numpy as jnp
from jax import lax
from jax.experimental import pallas as pl
from jax.experimental.pallas import tpu as pltpu
from jax.experimental.pallas import tpu_sc as plsc

N = 10000
K = 32
C_IN = 128
C_OUT = 128
C_MID = 32
KP = 15
SIGMA = 2.0
GROUPS = 8
EPS = 1e-5
NEG_SLOPE = 0.1

D_TAB = 48
BM = 200


def _leaky(x):
    return jnp.where(x >= 0, x, NEG_SLOPE * x)


def _group_indicator(c, dtype):
    per_g = c // GROUPS
    row_g = lax.broadcasted_iota(jnp.int32, (c, GROUPS), 0) // per_g
    col = lax.broadcasted_iota(jnp.int32, (c, GROUPS), 1)
    return (row_g == col).astype(dtype)


def _group_norm(x, gamma, beta, n_rows):
    c = x.shape[1]
    ind = _group_indicator(c, x.dtype)
    cs = jnp.sum(x, axis=0, keepdims=True)
    css = jnp.sum(x * x, axis=0, keepdims=True)
    cnt = float(n_rows * (c // GROUPS))
    gmean = jnp.dot(cs, ind, preferred_element_type=jnp.float32) / cnt
    gmsq = jnp.dot(css, ind, preferred_element_type=jnp.float32) / cnt
    var = gmsq - gmean * gmean
    rstd = lax.rsqrt(var + EPS)
    mean_c = jnp.dot(gmean, ind.T, preferred_element_type=jnp.float32)
    rstd_c = jnp.dot(rstd, ind.T, preferred_element_type=jnp.float32)
    return (x - mean_c) * rstd_c * gamma + beta


def _unary1_body(x_ref, p_ref, w_ref, g_ref, b_ref, o_ref):
    xm = jnp.dot(x_ref[...], w_ref[...], preferred_element_type=jnp.float32)
    y1 = _leaky(_group_norm(xm, g_ref[...], b_ref[...], N))
    o_ref[:, 0:C_MID] = y1
    o_ref[:, C_MID:C_MID + 3] = p_ref[...]
    rs = jnp.sum(y1, axis=1, keepdims=True)
    o_ref[:, C_MID + 3:C_MID + 4] = (rs != 0).astype(jnp.float32)
    o_ref[:, C_MID + 4:D_TAB] = jnp.zeros((N, D_TAB - C_MID - 4), jnp.float32)


def _unary1(s_feats, s_points, W1, gamma1, beta1):
    return pl.pallas_call(
        _unary1_body,
        out_shape=jax.ShapeDtypeStruct((N, D_TAB), jnp.float32),
    )(s_feats, s_points, W1, gamma1.reshape(1, C_MID), beta1.reshape(1, C_MID))


def _sc_gather(table, idx_flat):
    info = plsc.get_sparse_core_info()
    nw = info.num_cores * info.num_subcores
    tot = N * K
    per_w = tot // nw
    ch = 1000
    n_ch = per_w // ch
    mesh = plsc.VectorSubcoreMesh(core_axis_name="c", subcore_axis_name="s")

    @functools.partial(
        pl.kernel,
        mesh=mesh,
        compiler_params=pltpu.CompilerParams(use_tc_tiling_on_sc=False),
        out_type=jax.ShapeDtypeStruct((K, N, D_TAB), jnp.float32),
        scratch_types=[
            pltpu.VMEM((ch,), jnp.int32),
            pltpu.VMEM((ch, D_TAB), jnp.float32),
            pltpu.SemaphoreType.DMA,
        ],
    )
    def gather(table_hbm, idx_hbm, out_hbm, idx_v, rows_v, sem):
        wid = lax.axis_index("s") * info.num_cores + lax.axis_index("c")
        base = wid * per_w

        def body(j, carry):
            off = base + j * ch
            pltpu.sync_copy(idx_hbm.at[pl.ds(off, ch)], idx_v)
            pltpu.async_copy(table_hbm.at[idx_v], rows_v, sem).wait()
            pltpu.sync_copy(rows_v, out_hbm.at[wid, pl.ds(j * ch, ch)])
            return carry

        lax.fori_loop(0, n_ch, body, 0)

    return gather(table, idx_flat)


def _kpconv_body(q_ref, kpt_ref, r_ref, t_ref, w_ref, g_hbm, o_ref, gbuf, sems):
    i = pl.program_id(0)
    nb = pl.num_programs(0)

    def win_copy(blk, slot):
        return pltpu.make_async_copy(
            g_hbm.at[:, pl.ds(blk * BM, BM), :], gbuf.at[slot], sems.at[slot])

    @pl.when(i == 0)
    def _():
        win_copy(0, 0).start()

    @pl.when(i + 1 < nb)
    def _():
        win_copy(i + 1, (i + 1) % 2).start()

    win_copy(i, i % 2).wait()
    g = gbuf[i % 2]

    q = q_ref[...]
    kpt = kpt_ref[...]
    rmat = r_ref[...]
    tmat = t_ref[...]
    wf = jnp.zeros((BM, KP * C_MID), jnp.float32)
    cnt = jnp.zeros((BM, 1), jnp.float32)
    for k in range(K):
        gk = g[k]
        nf_k = gk[:, 0:C_MID]
        pk = gk[:, C_MID:C_MID + 3]
        r = pk - q
        dx = r[:, 0:1] - kpt[0:1, :]
        dy = r[:, 1:2] - kpt[1:2, :]
        dz = r[:, 2:3] - kpt[2:3, :]
        d2 = dx * dx + dy * dy + dz * dz
        w = jnp.maximum(1.0 - jnp.sqrt(d2) * (1.0 / SIGMA), 0.0)
        wrep = jnp.dot(w, rmat, preferred_element_type=jnp.float32)
        nfrep = jnp.dot(nf_k, tmat, preferred_element_type=jnp.float32)
        wf = wf + wrep * nfrep
        cnt = cnt + gk[:, C_MID + 3:C_MID + 4]
    out = jnp.dot(wf, w_ref[...], preferred_element_type=jnp.float32)
    o_ref[...] = out / jnp.maximum(cnt, 1.0)


def _kpconv(g3, q_points, kp_t, r_mat, t_mat, w_flat):
    n_blocks = N // BM
    return pl.pallas_call(
        _kpconv_body,
        grid=(n_blocks,),
        in_specs=[
            pl.BlockSpec((BM, 3), lambda i: (i, 0)),
            pl.BlockSpec((3, KP), lambda i: (0, 0)),
            pl.BlockSpec((KP, KP * C_MID), lambda i: (0, 0)),
            pl.BlockSpec((C_MID, KP * C_MID), lambda i: (0, 0)),
            pl.BlockSpec((KP * C_MID, C_MID), lambda i: (0, 0)),
            pl.BlockSpec(memory_space=pl.ANY),
        ],
        out_specs=pl.BlockSpec((BM, C_MID), lambda i: (i, 0)),
        out_shape=jax.ShapeDtypeStruct((N, C_MID), jnp.float32),
        scratch_shapes=[
            pltpu.VMEM((2, K, BM, D_TAB), jnp.float32),
            pltpu.SemaphoreType.DMA((2,)),
        ],
    )(q_points, kp_t, r_mat, t_mat, w_flat, g3)


def _tail_body(x_ref, f_ref, w2_ref, gc_ref, bc_ref, g2_ref, b2_ref, o_ref):
    x = _leaky(_group_norm(x_ref[...], gc_ref[...], bc_ref[...], N))
    y = jnp.dot(x, w2_ref[...], preferred_element_type=jnp.float32)
    y = _group_norm(y, g2_ref[...], b2_ref[...], N)
    o_ref[...] = _leaky(y + f_ref[...])


def _tail(out1, s_feats, W2, gamma_c, beta_c, gamma2, beta2):
    return pl.pallas_call(
        _tail_body,
        out_shape=jax.ShapeDtypeStruct((N, C_OUT), jnp.float32),
    )(out1, s_feats, W2,
      gamma_c.reshape(1, C_MID), beta_c.reshape(1, C_MID),
      gamma2.reshape(1, C_OUT), beta2.reshape(1, C_OUT))


def kernel(q_points, s_points, s_feats, neighbor_indices, kernel_points,
           W1, gamma1, beta1, conv_W, gamma_c, beta_c, W2, gamma2, beta2):
    table = _unary1(s_feats, s_points, W1, gamma1, beta1)
    idx_flat = neighbor_indices.astype(jnp.int32).T.reshape(-1)
    g3 = _sc_gather(table, idx_flat)
    w_flat = conv_W.reshape(KP * C_MID, C_MID)
    kp_t = kernel_points.T
    j = jnp.arange(KP * C_MID)
    r_mat = (jnp.arange(KP)[:, None] == (j[None, :] // C_MID)).astype(jnp.float32)
    t_mat = (jnp.arange(C_MID)[:, None] == (j[None, :] % C_MID)).astype(jnp.float32)
    out1 = _kpconv(g3, q_points, kp_t, r_mat, t_mat, w_flat)
    return _tail(out1, s_feats, W2, gamma_c, beta_c, gamma2, beta2)
